# baseline (speedup 1.0000x reference)
import functools

import jax
import jax.numpy as jnp
from jax import lax
from jax.experimental import pallas as pl
from jax.experimental.pallas import tpu as pltpu, tpu_sc as plsc

VOCAB = 1000000
D = 64
NB, NS_SEQ = 4096, 200
B = NB * NS_SEQ
NC, NS = 2, 16
NW = NC * NS
NB_PER_W = NB // NW
B_PER_W = B // NW
CHUNK = 40
CPB = NS_SEQ // CHUNK
NCHUNK = B_PER_W // CHUNK
R = 3
L = 16

_mesh = plsc.VectorSubcoreMesh(
    core_axis_name="c", subcore_axis_name="s", num_cores=NC, num_subcores=NS
)


@functools.partial(
    pl.kernel,
    out_type=jax.ShapeDtypeStruct((NB, NS_SEQ, D), jnp.float32),
    mesh=_mesh,
    scratch_types=[
        pltpu.VMEM((B_PER_W + L,), jnp.int32),
        pltpu.VMEM((R, 48), jnp.int32),
        pltpu.VMEM((R, 48), jnp.int32),
        pltpu.VMEM((R, CHUNK, 2 * D), jnp.float32),
        pltpu.VMEM((R, CHUNK, D), jnp.float32),
        pltpu.SemaphoreType.DMA,
        pltpu.SemaphoreType.DMA,
        pltpu.SemaphoreType.DMA,
        pltpu.SemaphoreType.DMA,
        pltpu.SemaphoreType.DMA,
        pltpu.SemaphoreType.DMA,
    ],
)
def _emb_lookup(idx_hbm, table_hbm, out_hbm, xv, qbuf, obuf, pairs_v, rows_v,
                g0, g1, g2, s0, s1, s2):
    wid = lax.axis_index("s") * NC + lax.axis_index("c")
    base = wid * B_PER_W
    b_base = wid * NB_PER_W
    gsem = (g0, g1, g2)
    ssem = (s0, s1, s2)

    pltpu.sync_copy(idx_hbm.at[pl.ds(base, B_PER_W)], xv.at[pl.ds(0, B_PER_W)])

    def fire_gather(gi, r):
        for j in range((CHUNK + L - 1) // L):
            v = xv[pl.ds(gi * CHUNK + j * L, L)]
            obuf[r, pl.ds(j * L, L)] = (v & 1) << 6
            qbuf[r, pl.ds(j * L, L)] = v >> 1

        pltpu.make_async_copy(
            table_hbm.at[qbuf.at[r, pl.ds(0, CHUNK)]], pairs_v.at[r], gsem[r]
        ).start()

    def gather_wait(r):
        pltpu.make_async_copy(
            table_hbm.at[qbuf.at[r, pl.ds(0, CHUNK)]], pairs_v.at[r], gsem[r]
        ).wait()

    def store_desc(gi, r):
        bb = b_base + gi // CPB
        s0_ = (gi % CPB) * CHUNK
        out_sl = out_hbm.at[bb, pl.ds(s0_, CHUNK), :]
        return pltpu.make_async_copy(rows_v.at[r], out_sl, ssem[r])

    def extract(r):
        ovecs = [obuf[r, pl.ds(k * L, L)] for k in range((CHUNK + L - 1) // L)]
        for i in range(CHUNK):
            off = ovecs[i // L][i % L]
            for k in range(D // L):
                rows_v[r, i, pl.ds(k * L, L)] = pairs_v[r, i, pl.ds(off + k * L, L)]

    fire_gather(0, 0)

    @pl.loop(0, NCHUNK - 1, step=3)
    def _steady(i):
        for p in range(3):
            g = i + p
            r = p
            rn = (r + 1) % R

            @pl.when(g >= 2)
            def _drain():
                store_desc(g - 2, rn).wait()

            fire_gather(g + 1, rn)
            gather_wait(r)
            extract(r)
            store_desc(g, r).start()

    store_desc(NCHUNK - 3, 1).wait()
    gather_wait(0)
    extract(0)
    store_desc(NCHUNK - 1, 0).start()

    store_desc(NCHUNK - 2, 2).wait()
    store_desc(NCHUNK - 1, 0).wait()


def kernel(x, table):
    table2 = table.reshape(VOCAB // 2, 2 * D)
    return _emb_lookup(x.reshape(-1), table2)
